# baseline (speedup 1.0000x reference)
import functools

import jax
import jax.numpy as jnp
from jax import lax
from jax.experimental import pallas as pl
from jax.experimental.pallas import tpu as pltpu
from jax.experimental.pallas import tpu_sc as plsc

NC = 2
NS = 16
NW = NC * NS

NUM_ROWS = 4096 * 50
EMBED_DIM = 512
B_PER_W = NUM_ROWS // NW
CHUNK = 64
N_CHUNK = B_PER_W // CHUNK


def _sc_body(table_hbm, idx_hbm, out_hbm, idx_v, rows_v, gsem0, gsem1):
    wid = lax.axis_index("s") * NC + lax.axis_index("c")
    base = wid * B_PER_W

    pltpu.sync_copy(idx_hbm.at[wid], idx_v)

    gsems = (gsem0, gsem1)

    pltpu.async_copy(table_hbm.at[idx_v.at[0]], rows_v.at[0], gsem0)
    pltpu.async_copy(table_hbm.at[idx_v.at[1]], rows_v.at[1], gsem1)

    @pl.loop(0, N_CHUNK, step=2)
    def _(c):
        for b in range(2):
            cc = c + b
            pltpu.make_async_copy(
                table_hbm.at[idx_v.at[cc]], rows_v.at[b], gsems[b]
            ).wait()
            pltpu.sync_copy(
                rows_v.at[b], out_hbm.at[pl.ds(base + cc * CHUNK, CHUNK)]
            )
            @pl.when(cc + 2 < N_CHUNK)
            def _():
                pltpu.async_copy(
                    table_hbm.at[idx_v.at[cc + 2]], rows_v.at[b], gsems[b]
                )


@jax.jit
def _sc_lookup(table, idx3):
    mesh = plsc.VectorSubcoreMesh(
        core_axis_name="c", subcore_axis_name="s", num_cores=NC, num_subcores=NS
    )
    return pl.kernel(
        _sc_body,
        out_type=jax.ShapeDtypeStruct((NUM_ROWS, EMBED_DIM), jnp.float32),
        mesh=mesh,
        scratch_types=[
            pltpu.VMEM((N_CHUNK, CHUNK), jnp.int32),
            pltpu.VMEM((2, CHUNK, EMBED_DIM), jnp.float32),
            pltpu.SemaphoreType.DMA,
            pltpu.SemaphoreType.DMA,
        ],
    )(table, idx3)


def kernel(x, table):
    idx3 = x.astype(jnp.int32).reshape(NW, N_CHUNK, CHUNK)
    out = _sc_lookup(table, idx3)
    return out.reshape(x.shape[0], x.shape[1], EMBED_DIM)


# speedup vs baseline: 2.4212x; 2.4212x over previous
import functools

import jax
import jax.numpy as jnp
from jax import lax
from jax.experimental import pallas as pl
from jax.experimental.pallas import tpu as pltpu
from jax.experimental.pallas import tpu_sc as plsc

NC = 2
NS = 16
NW = NC * NS

NUM_ROWS = 4096 * 50
EMBED_DIM = 512
B_PER_W = NUM_ROWS // NW
CHUNK = 64
N_CHUNK = B_PER_W // CHUNK


def _sc_body(table_hbm, idx_hbm, out_hbm, idx_v, rows_v, gsem0, gsem1):
    sid = lax.axis_index("s")
    wid = sid * NC + lax.axis_index("c")
    base = wid * B_PER_W

    pltpu.sync_copy(idx_hbm.at[wid], idx_v)

    gsems = (gsem0, gsem1)

    pltpu.async_copy(table_hbm.at[idx_v.at[0]], rows_v.at[0], gsem0)
    pltpu.async_copy(table_hbm.at[idx_v.at[1]], rows_v.at[1], gsem1)

    @pl.loop(0, N_CHUNK, step=2)
    def _(c):
        for b in range(2):
            cc = c + b
            pltpu.make_async_copy(
                table_hbm.at[idx_v.at[cc]], rows_v.at[b], gsems[b]
            ).wait()
            pltpu.sync_copy(
                rows_v.at[b], out_hbm.at[pl.ds(base + cc * CHUNK, CHUNK)]
            )
            @pl.when(cc + 2 < N_CHUNK)
            def _():
                pltpu.async_copy(
                    table_hbm.at[idx_v.at[cc + 2]], rows_v.at[b], gsems[b]
                )


@jax.jit
def _sc_lookup(table, idx3):
    mesh = plsc.VectorSubcoreMesh(
        core_axis_name="c", subcore_axis_name="s", num_cores=NC, num_subcores=NS
    )
    return pl.kernel(
        _sc_body,
        out_type=jax.ShapeDtypeStruct((NUM_ROWS, EMBED_DIM), jnp.float32),
        mesh=mesh,
        scratch_types=[
            pltpu.VMEM((N_CHUNK, CHUNK), jnp.int32),
            pltpu.VMEM((2, CHUNK, EMBED_DIM), jnp.float32),
            pltpu.SemaphoreType.DMA,
            pltpu.SemaphoreType.DMA,
        ],
    )(table, idx3)


def kernel(x, table):
    idx3 = x.astype(jnp.int32).reshape(NW, N_CHUNK, CHUNK)
    idx3 = idx3 + (4 * jnp.arange(NW, dtype=jnp.int32))[:, None, None]
    table_rep = jnp.tile(table, (NW, 1))
    out = _sc_lookup(table_rep, idx3)
    return out.reshape(x.shape[0], x.shape[1], EMBED_DIM)
